# baseline (speedup 1.0000x reference)
import math

import jax
import jax.numpy as jnp
from jax import lax
from jax.experimental import pallas as pl
from jax.experimental.pallas import tpu as pltpu
from jax.experimental.pallas import tpu_sc as plsc

D = 32
SCALE = math.sqrt(32.0)
PITCH3 = 513

NW = 32
CHUNK = 128
NCH = 8
TASK_B = 512
NSTR = TASK_B // CHUNK


def _emb_body(tok_hbm, tab0_hbm, tab1_hbm, out_hbm, idx_all, buf_a0,
              buf_a1, buf_b0, buf_b1, buf3, buf2_a, buf2_b, gsem_a, gsem_b,
              osem_a, osem_b, *, n_t, n_b):
    ntasks = n_t * NCH
    per_w = ntasks // NW
    wid = lax.axis_index("s") * 2 + lax.axis_index("c")
    k0 = wid * per_w
    t_words = n_b * D

    pltpu.sync_copy(tok_hbm.at[pl.ds(k0 * NSTR, per_w * NSTR)], idx_all)

    iota = lax.iota(jnp.int32, 16)

    def fire(kl, bufs, gsem):
        for h, tab in ((0, tab0_hbm), (1, tab1_hbm)):
            for ci in range(NSTR):
                pltpu.async_copy(
                    tab.at[idx_all.at[kl * NSTR + ci]],
                    bufs[h].at[pl.ds(ci * CHUNK, CHUNK)],
                    gsem,
                )

    def drain_gather(bufs, gsem):
        pltpu.make_async_copy(
            tab0_hbm.at[pl.ds(0, TASK_B)], bufs[0], gsem
        ).wait()
        pltpu.make_async_copy(
            tab1_hbm.at[pl.ds(0, TASK_B)], bufs[1], gsem
        ).wait()

    def drain_stores(buf2, osem):
        pltpu.make_async_copy(
            out_hbm.at[0, pl.ds(0, TASK_B * D)], buf2, osem
        ).wait()

    def transpose_scale(bufs, buf3, buf2):
        for dh in range(2):
            pb = (dh * 16 + iota) * PITCH3

            bufh = bufs[dh]

            @plsc.parallel_loop(0, TASK_B, unroll=8)
            def _(bl, pb=pb, bufh=bufh):
                v = bufh[bl, pl.ds(0, 16)]
                plsc.store_scatter(buf3, [pb + bl], v * SCALE)

        @plsc.parallel_loop(0, D * (TASK_B // 16), unroll=8)
        def _(i):
            d = i >> 5
            lg = i & 31
            v = buf3[pl.ds(d * PITCH3 + lg * 16, 16)]
            dst = ((d >> 3) * 4096 + (lg >> 3) * 1024
                   + (d & 7) * 128 + (lg & 7) * 16)
            buf2[pl.ds(dst, 16)] = v

    def stores(kl, buf2, osem):
        k = k0 + kl
        t = k // NCH
        c8 = k % NCH
        for r in range(4):
            pltpu.async_copy(
                buf2.at[pl.ds(r * (NSTR * 1024), NSTR * 1024)],
                out_hbm.at[
                    t, pl.ds(r * (t_words // 4) + c8 * (NSTR * 1024),
                             NSTR * 1024)
                ],
                osem,
            )

    npairs = per_w // 2
    fire(0, (buf_a0, buf_a1), gsem_a)

    def pair(p, _):
        kl0 = 2 * p
        fire(kl0 + 1, (buf_b0, buf_b1), gsem_b)
        drain_gather((buf_a0, buf_a1), gsem_a)

        @pl.when(p > 0)
        def _():
            drain_stores(buf2_a, osem_a)

        transpose_scale((buf_a0, buf_a1), buf3, buf2_a)
        stores(kl0, buf2_a, osem_a)

        @pl.when(p + 1 < npairs)
        def _():
            fire(kl0 + 2, (buf_a0, buf_a1), gsem_a)

        drain_gather((buf_b0, buf_b1), gsem_b)

        @pl.when(p > 0)
        def _():
            drain_stores(buf2_b, osem_b)

        transpose_scale((buf_b0, buf_b1), buf3, buf2_b)
        stores(kl0 + 1, buf2_b, osem_b)
        return 0

    lax.fori_loop(0, npairs, pair, 0)
    drain_stores(buf2_a, osem_a)
    drain_stores(buf2_b, osem_b)


def kernel(tokens, table):
    n_b, n_t = tokens.shape
    ntasks = n_t * NCH
    per_w = ntasks // NW
    tok2 = tokens.T.reshape(ntasks * NSTR, CHUNK).astype(jnp.int32)
    t_words = n_b * D

    mesh = plsc.VectorSubcoreMesh(core_axis_name="c", subcore_axis_name="s")
    run = pl.kernel(
        lambda *a: _emb_body(*a, n_t=n_t, n_b=n_b),
        out_type=jax.ShapeDtypeStruct((n_t, t_words), jnp.float32),
        mesh=mesh,
        scratch_types=[
            pltpu.VMEM((per_w * NSTR, CHUNK), jnp.int32),
            pltpu.VMEM((TASK_B, D // 2), jnp.float32),
            pltpu.VMEM((TASK_B, D // 2), jnp.float32),
            pltpu.VMEM((TASK_B, D // 2), jnp.float32),
            pltpu.VMEM((TASK_B, D // 2), jnp.float32),
            pltpu.VMEM((D * PITCH3,), jnp.float32),
            pltpu.VMEM((TASK_B * D,), jnp.float32),
            pltpu.VMEM((TASK_B * D,), jnp.float32),
            pltpu.SemaphoreType.DMA,
            pltpu.SemaphoreType.DMA,
            pltpu.SemaphoreType.DMA,
            pltpu.SemaphoreType.DMA,
        ],
        compiler_params=pltpu.CompilerParams(
            use_tc_tiling_on_sc=False, needs_layout_passes=False
        ),
    )
    out2 = run(tok2, table[:, : D // 2], table[:, D // 2:])
    out5 = out2.reshape(n_t, 4, n_b // CHUNK, 8, CHUNK)
    return out5.transpose(2, 4, 0, 1, 3).reshape(n_b, n_t, D)


# speedup vs baseline: 2.2365x; 2.2365x over previous
import math

import jax
import jax.numpy as jnp
from jax import lax
from jax.experimental import pallas as pl
from jax.experimental.pallas import tpu as pltpu
from jax.experimental.pallas import tpu_sc as plsc

D = 32
SCALE = math.sqrt(32.0)
PITCH3 = 513

NW = 32
CHUNK = 128
NCH = 8
TASK_B = 512
NSTR = TASK_B // CHUNK


def _emb_body(tok_hbm, table_hbm, out_hbm, idx_all, buf_a, buf_b, buf3,
              buf2_a, buf2_b, gsem_a, gsem_b, osem_a, osem_b, *, n_t, n_b):
    ntasks = n_t * NCH
    per_w = ntasks // NW
    wid = lax.axis_index("s") * 2 + lax.axis_index("c")
    k0 = wid * per_w
    t_words = n_b * D

    pltpu.sync_copy(tok_hbm.at[pl.ds(k0 * NSTR, per_w * NSTR)], idx_all)

    iota = lax.iota(jnp.int32, 16)

    def fire(kl, buf, gsem):
        for ci in range(NSTR):
            pltpu.async_copy(
                table_hbm.at[idx_all.at[kl * NSTR + ci]],
                buf.at[pl.ds(ci * CHUNK, CHUNK)],
                gsem,
            )

    def drain_gather(buf, gsem):
        pltpu.make_async_copy(
            table_hbm.at[pl.ds(0, TASK_B)], buf, gsem
        ).wait()

    def drain_stores(buf2, osem):
        pltpu.make_async_copy(
            out_hbm.at[0, pl.ds(0, TASK_B * D)], buf2, osem
        ).wait()

    def transpose_scale(buf, buf3, buf2):
        for dh in range(2):
            pb = (dh * 16 + iota) * PITCH3

            @plsc.parallel_loop(0, TASK_B, unroll=8)
            def _(bl, pb=pb, dh=dh):
                v = buf[bl, pl.ds(dh * 16, 16)]
                plsc.store_scatter(buf3, [pb + bl], v * SCALE)

        @plsc.parallel_loop(0, D * (TASK_B // 16), unroll=8)
        def _(i):
            d = i >> 5
            lg = i & 31
            v = buf3[pl.ds(d * PITCH3 + lg * 16, 16)]
            dst = ((d >> 3) * 4096 + (lg >> 3) * 1024
                   + (d & 7) * 128 + (lg & 7) * 16)
            buf2[pl.ds(dst, 16)] = v

    def stores(kl, buf2, osem):
        k = k0 + kl
        t = k // NCH
        c8 = k % NCH
        for r in range(4):
            pltpu.async_copy(
                buf2.at[pl.ds(r * (NSTR * 1024), NSTR * 1024)],
                out_hbm.at[
                    t, pl.ds(r * (t_words // 4) + c8 * (NSTR * 1024),
                             NSTR * 1024)
                ],
                osem,
            )

    npairs = per_w // 2
    fire(0, buf_a, gsem_a)

    def pair(p, _):
        kl0 = 2 * p
        fire(kl0 + 1, buf_b, gsem_b)
        drain_gather(buf_a, gsem_a)

        @pl.when(p > 0)
        def _():
            drain_stores(buf2_a, osem_a)

        transpose_scale(buf_a, buf3, buf2_a)
        stores(kl0, buf2_a, osem_a)

        @pl.when(p + 1 < npairs)
        def _():
            fire(kl0 + 2, buf_a, gsem_a)

        drain_gather(buf_b, gsem_b)

        @pl.when(p > 0)
        def _():
            drain_stores(buf2_b, osem_b)

        transpose_scale(buf_b, buf3, buf2_b)
        stores(kl0 + 1, buf2_b, osem_b)
        return 0

    lax.fori_loop(0, npairs, pair, 0)
    drain_stores(buf2_a, osem_a)
    drain_stores(buf2_b, osem_b)


def kernel(tokens, table):
    n_b, n_t = tokens.shape
    ntasks = n_t * NCH
    per_w = ntasks // NW
    tok2 = tokens.T.reshape(ntasks * NSTR, CHUNK).astype(jnp.int32)
    t_words = n_b * D

    mesh = plsc.VectorSubcoreMesh(core_axis_name="c", subcore_axis_name="s")
    run = pl.kernel(
        lambda *a: _emb_body(*a, n_t=n_t, n_b=n_b),
        out_type=jax.ShapeDtypeStruct((n_t, t_words), jnp.float32),
        mesh=mesh,
        scratch_types=[
            pltpu.VMEM((per_w * NSTR, CHUNK), jnp.int32),
            pltpu.VMEM((TASK_B, D), jnp.float32),
            pltpu.VMEM((TASK_B, D), jnp.float32),
            pltpu.VMEM((D * PITCH3,), jnp.float32),
            pltpu.VMEM((TASK_B * D,), jnp.float32),
            pltpu.VMEM((TASK_B * D,), jnp.float32),
            pltpu.SemaphoreType.DMA,
            pltpu.SemaphoreType.DMA,
            pltpu.SemaphoreType.DMA,
            pltpu.SemaphoreType.DMA,
        ],
        compiler_params=pltpu.CompilerParams(
            use_tc_tiling_on_sc=False, needs_layout_passes=False
        ),
    )
    out2 = run(tok2, table)
    out5 = out2.reshape(n_t, 4, n_b // CHUNK, 8, CHUNK)
    return out5.transpose(2, 4, 0, 1, 3).reshape(n_b, n_t, D)
